# baseline (speedup 1.0000x reference)
import functools

import jax
import jax.numpy as jnp
from jax import lax
from jax.experimental import pallas as pl
from jax.experimental.pallas import tpu as pltpu
from jax.experimental.pallas import tpu_sc as plsc

_TB = 2048


def _widen_body(x_ref, o_ref):
    D = x_ref.shape[0]
    rows = lax.broadcasted_iota(jnp.int32, (D, D), 0)
    cols = lax.broadcasted_iota(jnp.int32, (D, D), 1)
    eye = (rows == cols).astype(jnp.float32)
    o_ref[:, 0:D] = lax.dot_general(
        x_ref[...], eye, (((0,), (0,)), ((), ())),
        preferred_element_type=jnp.float32,
    )


def _tc_table_widen(table):
    V, D = table.shape
    nblk = pl.cdiv(V, _TB)
    return pl.pallas_call(
        _widen_body,
        grid=(nblk,),
        in_specs=[pl.BlockSpec((D, _TB), lambda j: (0, j))],
        out_specs=pl.BlockSpec((_TB, 128), lambda j: (j, 0)),
        out_shape=jax.ShapeDtypeStruct((V, 128), jnp.float32),
    )(table.T)


def _sc_gather(table128, input_id):
    B = input_id.shape[0]
    W = table128.shape[1]
    info = plsc.get_sparse_core_info()
    NW = info.num_cores * info.num_subcores
    b_per_w = B // NW
    mesh = plsc.VectorSubcoreMesh(core_axis_name="c", subcore_axis_name="s")

    @functools.partial(
        pl.kernel,
        mesh=mesh,
        out_type=jax.ShapeDtypeStruct((B, W), jnp.float32),
        scratch_types=[
            pltpu.VMEM((b_per_w,), jnp.int32),
            pltpu.VMEM((b_per_w, W), jnp.float32),
            pltpu.SemaphoreType.DMA,
        ],
        compiler_params=pltpu.CompilerParams(use_tc_tiling_on_sc=False),
    )
    def gather_kernel(table_hbm, idx_hbm, out_hbm, idx_v, rows_v, sem):
        wid = lax.axis_index("s") * info.num_cores + lax.axis_index("c")
        base = wid * b_per_w
        pltpu.sync_copy(idx_hbm.at[pl.ds(base, b_per_w)], idx_v)
        pltpu.async_copy(table_hbm.at[idx_v], rows_v, sem).wait()
        pltpu.sync_copy(rows_v, out_hbm.at[pl.ds(base, b_per_w)])

    return gather_kernel(table128, input_id.astype(jnp.int32))


_VB = 4096


def _contract(lhs, rhs):
    return lax.dot_general(
        lhs, rhs, (((0,), (1,)), ((), ())), preferred_element_type=jnp.float32
    )


def _classifier_body(vec_ref, wct_ref, bc_ref, wit_ref, bi_ref,
                     vect_ref, catt_ref, idt_ref):
    vec = vec_ref[:, 0:64]

    @pl.when(pl.program_id(0) == 0)
    def _():
        D = vec.shape[1]
        rows = lax.broadcasted_iota(jnp.int32, (D, D), 0)
        cols = lax.broadcasted_iota(jnp.int32, (D, D), 1)
        eye = (rows == cols).astype(jnp.float32)
        vect_ref[...] = _contract(eye, vec)
        catt_ref[...] = _contract(wct_ref[...], vec) + bc_ref[...]

    bi_col = jnp.swapaxes(bi_ref[...].reshape(1, -1), 0, 1)
    idt_ref[...] = _contract(wit_ref[...], vec) + bi_col


def _tc_classifiers(vec128, Wc, bc, Wi, bi):
    B = vec128.shape[0]
    C, D = Wc.shape
    V = Wi.shape[0]
    nblk = pl.cdiv(V, _VB)
    vect, catt, idt = pl.pallas_call(
        _classifier_body,
        grid=(nblk,),
        in_specs=[
            pl.BlockSpec((B, 128), lambda j: (0, 0)),
            pl.BlockSpec((D, C), lambda j: (0, 0)),
            pl.BlockSpec((C, 1), lambda j: (0, 0)),
            pl.BlockSpec((D, _VB), lambda j: (0, j)),
            pl.BlockSpec((_VB,), lambda j: (j,)),
        ],
        out_specs=[
            pl.BlockSpec((D, B), lambda j: (0, 0)),
            pl.BlockSpec((C, B), lambda j: (0, 0)),
            pl.BlockSpec((_VB, B), lambda j: (j, 0)),
        ],
        out_shape=[
            jax.ShapeDtypeStruct((D, B), jnp.float32),
            jax.ShapeDtypeStruct((C, B), jnp.float32),
            jax.ShapeDtypeStruct((V, B), jnp.float32),
        ],
    )(vec128, Wc.T, bc.reshape(C, 1), Wi.T, bi)
    return vect.T, catt.T, idt.T


def kernel(table, Wc, bc, Wi, bi, input_id):
    vec128 = _sc_gather(_tc_table_widen(table), input_id)
    vec_out, out_category, out_identity = _tc_classifiers(vec128, Wc, bc, Wi, bi)
    return (vec_out, out_category, out_identity)
